# baseline (speedup 1.0000x reference)
import jax
import jax.numpy as jnp
from jax import lax
from jax.experimental import pallas as pl
from jax.experimental.pallas import tpu as pltpu
from jax.experimental.pallas import tpu_sc as plsc

LAM = 1.0
NC = 2
NS = 16
NW = NC * NS
LANES = 16
CHUNK = 128
CPW = 50
EW = CHUNK * CPW
EPAD = NW * EW
SUB = CHUNK // LANES
ROWS = 784


def _sc_body(table_hbm, conns_hbm, mu_hbm, out_hbm,
             acc, cv, mv, ridx, g, semA0, semA1, semB0, semB1):
    c = lax.axis_index("c")
    s = lax.axis_index("s")
    w = c * NS + s
    semA = (semA0, semA1)
    semB = (semB0, semB1)

    zero16 = jnp.zeros((LANES,), jnp.float32)

    def _zero(r, _):
        for jj in range(8):
            acc[r, pl.ds(jj * LANES, LANES)] = zero16
        return 0

    lax.fori_loop(0, ROWS, _zero, 0)

    base = w * EW
    lane = lax.iota(jnp.int32, LANES)

    def issueA(b, ci):
        e0 = base + ci * CHUNK
        pltpu.async_copy(conns_hbm.at[pl.ds(e0, CHUNK), :], cv.at[b], semA[b])
        pltpu.async_copy(mu_hbm.at[pl.ds(e0, CHUNK)], mv.at[b], semA[b])

    def waitA(b, ci):
        e0 = base + ci * CHUNK
        pltpu.make_async_copy(
            conns_hbm.at[pl.ds(e0, CHUNK), :], cv.at[b], semA[b]).wait()
        pltpu.make_async_copy(
            mu_hbm.at[pl.ds(e0, CHUNK)], mv.at[b], semA[b]).wait()

    def rowidx(b):
        for i in range(SUB):
            rows = i * LANES + lane
            for k in range(3):
                ik = plsc.load_gather(
                    cv.at[b], [rows, jnp.full((LANES,), k, jnp.int32)])
                ridx[b, k, pl.ds(i * LANES, LANES)] = ik >> 2

    def issueB(b):
        for k in range(3):
            pltpu.async_copy(table_hbm.at[ridx.at[b, k]], g.at[b, k], semB[b])

    def waitB(b):
        for k in range(3):
            pltpu.make_async_copy(
                table_hbm.at[ridx.at[b, k]], g.at[b, k], semB[b]).wait()

    def compute(b):
        for i in range(SUB):
            rows = i * LANES + lane
            ii = []
            vals = []
            for k in range(3):
                ik = plsc.load_gather(
                    cv.at[b], [rows, jnp.full((LANES,), k, jnp.int32)])
                col0 = (ik & 3) << 2
                xk = plsc.load_gather(g.at[b, k], [rows, col0])
                yk = plsc.load_gather(g.at[b, k], [rows, col0 + 1])
                uxk = plsc.load_gather(g.at[b, k], [rows, col0 + 2])
                uyk = plsc.load_gather(g.at[b, k], [rows, col0 + 3])
                ii.append(ik)
                vals.append((xk, yk, uxk, uyk))
            (x0, y0, ux0, uy0), (x1, y1, ux1, uy1), (x2, y2, ux2, uy2) = vals
            muv = mv[b, pl.ds(i * LANES, LANES)]

            detj = (x1 - x0) * (y2 - y0) - (y1 - y0) * (x2 - x0)
            safe = jnp.where(jnp.abs(detj) < 1e-6, jnp.float32(1e-6), detj)
            b0 = y1 - y2
            b1 = y2 - y0
            b2 = y0 - y1
            c0 = x2 - x1
            c1 = x0 - x2
            c2 = x1 - x0
            a = b0 * ux0 + b1 * ux1 + b2 * ux2
            bb = c0 * uy0 + c1 * uy1 + c2 * uy2
            cc = (b0 * uy0 + b1 * uy1 + b2 * uy2
                  + c0 * ux0 + c1 * ux1 + c2 * ux2)
            tr = a + bb
            w_ = 0.5 * LAM * tr * tr + muv * (a * a + bb * bb + 0.5 * cc * cc)
            contrib = w_ * jnp.abs(1.0 / safe) * jnp.float32(1.0 / 6.0)

            for k in range(3):
                plsc.addupdate_scatter(
                    acc, [ii[k] >> 7, ii[k] & 127], contrib)

    issueA(0, 0)
    waitA(0, 0)
    rowidx(0)
    issueB(0)
    issueA(1, 1)

    def _step(t, _):
        for b in range(2):
            ci = 2 * t + b
            waitA(1 - b, ci + 1)
            rowidx(1 - b)
            issueB(1 - b)
            waitB(b)
            compute(b)
            issueA(b, ci + 2)
        return 0

    lax.fori_loop(0, (CPW - 2) // 2, _step, 0)

    waitA(1, CPW - 1)
    rowidx(1)
    issueB(1)
    waitB(0)
    compute(0)
    waitB(1)
    compute(1)

    pltpu.sync_copy(acc, out_hbm.at[w])


def _tc_add(p_ref, o_ref):
    o_ref[...] = jnp.sum(p_ref[...], axis=0)


@jax.jit
def kernel(Uu, yLoc, mu, coords, conns, bc_nodes):
    n = coords.shape[0]
    e = conns.shape[0]
    n_bc = bc_nodes.shape[0]

    top = jnp.stack(
        [Uu[:n_bc], jnp.full((n_bc,), yLoc[0], jnp.float32)], axis=1)
    u_field = jnp.concatenate([top, Uu[n_bc:].reshape(-1, 2)], axis=0)
    table = jnp.concatenate([coords, u_field], axis=1)
    table4 = table.reshape(n // 4, 16)

    pad_idx = (jnp.arange(EPAD - e, dtype=jnp.int32) * 97) % n
    conns_pad = jnp.concatenate(
        [jnp.asarray(conns, jnp.int32),
         jnp.broadcast_to(pad_idx[:, None], (EPAD - e, 3))], axis=0)
    mu_pad = jnp.pad(mu, (0, EPAD - e))

    mesh = plsc.VectorSubcoreMesh(
        core_axis_name="c", subcore_axis_name="s",
        num_cores=NC, num_subcores=NS)
    sc = pl.kernel(
        _sc_body,
        out_type=jax.ShapeDtypeStruct((NW, ROWS, 128), jnp.float32),
        mesh=mesh,
        compiler_params=pltpu.CompilerParams(
            needs_layout_passes=False, use_tc_tiling_on_sc=False),
        scratch_types=[
            pltpu.VMEM((ROWS, 128), jnp.float32),
            pltpu.VMEM((2, CHUNK, 3), jnp.int32),
            pltpu.VMEM((2, CHUNK), jnp.float32),
            pltpu.VMEM((2, 3, CHUNK), jnp.int32),
            pltpu.VMEM((2, 3, CHUNK, 16), jnp.float32),
            pltpu.SemaphoreType.DMA,
            pltpu.SemaphoreType.DMA,
            pltpu.SemaphoreType.DMA,
            pltpu.SemaphoreType.DMA,
        ],
    )
    partial = sc(table4, conns_pad, mu_pad)

    total = pl.pallas_call(
        _tc_add,
        out_shape=jax.ShapeDtypeStruct((ROWS, 128), jnp.float32),
    )(partial)
    return total.reshape(ROWS * 128)[:n]


# speedup vs baseline: 1.3690x; 1.3690x over previous
import jax
import jax.numpy as jnp
from jax import lax
from jax.experimental import pallas as pl
from jax.experimental.pallas import tpu as pltpu
from jax.experimental.pallas import tpu_sc as plsc

LAM = 1.0
NC = 2
NS = 16
NW = NC * NS
LANES = 16
CHUNK = 128
CHUNKS_PER_W = 49
EW = CHUNK * CHUNKS_PER_W
EPAD = NW * EW
SUB = CHUNK // LANES


def _sc_body(table_hbm, connsT_hbm, mu_hbm, out_hbm,
             acc, idx_v, g, muc, sem):
    c = lax.axis_index("c")
    s = lax.axis_index("s")
    w = c * NS + s
    npad = acc.shape[0]

    zero16 = jnp.zeros((LANES,), jnp.float32)

    def _zero(i, _):
        acc[pl.ds(i * LANES, LANES)] = zero16
        return 0

    lax.fori_loop(0, npad // LANES, _zero, 0, unroll=8)

    base = w * EW
    lane = lax.iota(jnp.int32, LANES)

    def _chunk(gi, _):
        e0 = base + gi * CHUNK
        pltpu.sync_copy(connsT_hbm.at[:, pl.ds(e0, CHUNK)], idx_v)
        d_mu = pltpu.async_copy(mu_hbm.at[pl.ds(e0, CHUNK)], muc, sem)
        d0 = pltpu.async_copy(table_hbm.at[idx_v.at[0]], g.at[0], sem)
        d1 = pltpu.async_copy(table_hbm.at[idx_v.at[1]], g.at[1], sem)
        d2 = pltpu.async_copy(table_hbm.at[idx_v.at[2]], g.at[2], sem)
        d_mu.wait()
        d0.wait()
        d1.wait()
        d2.wait()

        for i in range(SUB):
            rows = i * LANES + lane
            cols = [jnp.full((LANES,), j, jnp.int32) for j in range(4)]
            x0 = plsc.load_gather(g.at[0], [rows, cols[0]])
            y0 = plsc.load_gather(g.at[0], [rows, cols[1]])
            ux0 = plsc.load_gather(g.at[0], [rows, cols[2]])
            uy0 = plsc.load_gather(g.at[0], [rows, cols[3]])
            x1 = plsc.load_gather(g.at[1], [rows, cols[0]])
            y1 = plsc.load_gather(g.at[1], [rows, cols[1]])
            ux1 = plsc.load_gather(g.at[1], [rows, cols[2]])
            uy1 = plsc.load_gather(g.at[1], [rows, cols[3]])
            x2 = plsc.load_gather(g.at[2], [rows, cols[0]])
            y2 = plsc.load_gather(g.at[2], [rows, cols[1]])
            ux2 = plsc.load_gather(g.at[2], [rows, cols[2]])
            uy2 = plsc.load_gather(g.at[2], [rows, cols[3]])
            muv = muc[pl.ds(i * LANES, LANES)]

            detj = (x1 - x0) * (y2 - y0) - (y1 - y0) * (x2 - x0)
            safe = jnp.where(jnp.abs(detj) < 1e-6, jnp.float32(1e-6), detj)
            b0 = y1 - y2
            b1 = y2 - y0
            b2 = y0 - y1
            c0 = x2 - x1
            c1 = x0 - x2
            c2 = x1 - x0
            a = b0 * ux0 + b1 * ux1 + b2 * ux2
            bb = c0 * uy0 + c1 * uy1 + c2 * uy2
            cc = (b0 * uy0 + b1 * uy1 + b2 * uy2
                  + c0 * ux0 + c1 * ux1 + c2 * ux2)
            tr = a + bb
            w_ = 0.5 * LAM * tr * tr + muv * (a * a + bb * bb + 0.5 * cc * cc)
            contrib = w_ * jnp.abs(1.0 / safe) * jnp.float32(1.0 / 6.0)

            i0 = plsc.load_gather(idx_v, [cols[0], rows])
            i1 = plsc.load_gather(idx_v, [cols[1], rows])
            i2 = plsc.load_gather(idx_v, [cols[2], rows])
            plsc.addupdate_scatter(acc, [i0], contrib)
            plsc.addupdate_scatter(acc, [i1], contrib)
            plsc.addupdate_scatter(acc, [i2], contrib)
        return 0

    lax.fori_loop(0, CHUNKS_PER_W, _chunk, 0)

    pltpu.sync_copy(acc, out_hbm.at[pl.ds(w * npad, npad)])


def _tc_add(p_ref, o_ref):
    o_ref[...] = jnp.sum(p_ref[...], axis=0)


@jax.jit
def kernel(Uu, yLoc, mu, coords, conns, bc_nodes):
    n = coords.shape[0]
    e = conns.shape[0]
    n_bc = bc_nodes.shape[0]
    npad = ((n + 1023) // 1024) * 1024

    top = jnp.stack(
        [Uu[:n_bc], jnp.full((n_bc,), yLoc[0], jnp.float32)], axis=1)
    u_field = jnp.concatenate([top, Uu[n_bc:].reshape(-1, 2)], axis=0)
    table = jnp.concatenate([coords, u_field], axis=1)
    table = jnp.pad(table, ((0, 0), (0, 12)))

    pad_idx = (jnp.arange(EPAD - e, dtype=jnp.int32) * 97) % n
    conns_pad = jnp.concatenate(
        [jnp.asarray(conns, jnp.int32),
         jnp.broadcast_to(pad_idx[:, None], (EPAD - e, 3))], axis=0)
    conns_t = conns_pad.T
    mu_pad = jnp.pad(mu, (0, EPAD - e))

    mesh = plsc.VectorSubcoreMesh(
        core_axis_name="c", subcore_axis_name="s",
        num_cores=NC, num_subcores=NS)
    sc = pl.kernel(
        _sc_body,
        out_type=jax.ShapeDtypeStruct((NW * npad,), jnp.float32),
        mesh=mesh,
        compiler_params=pltpu.CompilerParams(
            needs_layout_passes=False, use_tc_tiling_on_sc=False),
        scratch_types=[
            pltpu.VMEM((npad,), jnp.float32),
            pltpu.VMEM((3, CHUNK), jnp.int32),
            pltpu.VMEM((3, CHUNK, 16), jnp.float32),
            pltpu.VMEM((CHUNK,), jnp.float32),
            pltpu.SemaphoreType.DMA,
        ],
    )
    partial = sc(table, conns_t, mu_pad)

    p3 = partial.reshape(NW, npad // 128, 128)
    total = pl.pallas_call(
        _tc_add,
        out_shape=jax.ShapeDtypeStruct((npad // 128, 128), jnp.float32),
    )(p3)
    return total.reshape(npad)[:n]
